# baseline (speedup 1.0000x reference)
import functools

import jax
import jax.numpy as jnp
from jax import lax
from jax.experimental import pallas as pl
from jax.experimental.pallas import tpu as pltpu
from jax.experimental.pallas import tpu_sc as plsc

N_ROWS = 1000001
BATCH = 16384
EMBED_DIM = 64
NUM_CORES = 2
NUM_SUBCORES = 16
NUM_WORKERS = NUM_CORES * NUM_SUBCORES
B_PER_W = BATCH // NUM_WORKERS


def _transpose_body(tt_ref, eye_ref, out_ref):
    out_ref[...] = jax.lax.dot_general(
        tt_ref[...], eye_ref[...], (((0,), (0,)), ((), ())),
        preferred_element_type=jnp.float32,
    )


def _transpose_table(tt, col_chunk=512):
    eye = jnp.eye(EMBED_DIM, dtype=jnp.float32)
    grid = (pl.cdiv(N_ROWS, col_chunk),)
    return pl.pallas_call(
        _transpose_body,
        grid=grid,
        in_specs=[
            pl.BlockSpec((EMBED_DIM, col_chunk), lambda k: (0, k)),
            pl.BlockSpec((EMBED_DIM, EMBED_DIM), lambda k: (0, 0)),
        ],
        out_specs=pl.BlockSpec((col_chunk, EMBED_DIM), lambda k: (k, 0)),
        out_shape=jax.ShapeDtypeStruct((N_ROWS, EMBED_DIM), jnp.float32),
    )(tt, eye)


_mesh = plsc.VectorSubcoreMesh(core_axis_name="c", subcore_axis_name="s")


@functools.partial(
    pl.kernel,
    mesh=_mesh,
    out_type=jax.ShapeDtypeStruct((BATCH, EMBED_DIM), jnp.float32),
    scratch_types=[
        pltpu.VMEM((B_PER_W,), jnp.int32),
        pltpu.VMEM((B_PER_W, EMBED_DIM), jnp.float32),
        pltpu.SemaphoreType.DMA,
    ],
    compiler_params=pltpu.CompilerParams(use_tc_tiling_on_sc=True),
)
def _embed_gather(idx_hbm, table_hbm, out_hbm, idx_v, rows_v, sem):
    wid = lax.axis_index("s") * NUM_CORES + lax.axis_index("c")
    base = wid * B_PER_W
    pltpu.sync_copy(idx_hbm.at[pl.ds(base, B_PER_W)], idx_v)

    def issue(chunk, _):
        v = idx_v[pl.ds(chunk * 16, 16)]
        for j in range(16):
            pltpu.async_copy(table_hbm.at[v[j]], rows_v.at[chunk * 16 + j], sem)
        return _

    lax.fori_loop(0, B_PER_W // 16, issue, 0)

    def drain(i, _):
        pltpu.make_async_copy(table_hbm.at[0], rows_v.at[0], sem).wait()
        return _

    lax.fori_loop(0, B_PER_W, drain, 0)
    pltpu.sync_copy(rows_v, out_hbm.at[pl.ds(base, B_PER_W)])


def kernel(x, table):
    table_lin = _transpose_table(table.T)
    return _embed_gather(x.astype(jnp.int32), table_lin)


# speedup vs baseline: 1.9777x; 1.9777x over previous
import functools

import jax
import jax.numpy as jnp
from jax import lax
from jax.experimental import pallas as pl
from jax.experimental.pallas import tpu as pltpu
from jax.experimental.pallas import tpu_sc as plsc

N_ROWS = 1000001
BATCH = 16384
EMBED_DIM = 64
NUM_SUBCORES = 16
SLAB = BATCH // NUM_SUBCORES
W = 16384
BAND = W // NUM_SUBCORES
N_STRIPES = 61
TAIL_START = N_STRIPES * W

_mesh = plsc.VectorSubcoreMesh(core_axis_name="c", subcore_axis_name="s")


@functools.partial(
    pl.kernel,
    mesh=_mesh,
    out_type=jax.ShapeDtypeStruct((BATCH, EMBED_DIM), jnp.float32),
    scratch_types=[
        pltpu.VMEM((SLAB,), jnp.int32),
        pltpu.VMEM((SLAB,), jnp.int32),
        pltpu.VMEM((EMBED_DIM, BAND), jnp.float32),
        pltpu.VMEM((NUM_SUBCORES * 128,), jnp.int32),
        pltpu.VMEM((128,), jnp.int32),
        pltpu.VMEM((144,), jnp.int32),
        pltpu.VMEM((16, EMBED_DIM), jnp.float32),
        pltpu.VMEM_SHARED((NUM_SUBCORES * 128,), jnp.int32),
        pltpu.VMEM_SHARED((NUM_SUBCORES * SLAB,), jnp.int32),
        pltpu.SemaphoreType.DMA,
    ],
    compiler_params=pltpu.CompilerParams(use_tc_tiling_on_sc=True, needs_layout_passes=False),
)
def _embed(idx_hbm, tt_hbm, tail_hbm, out_hbm,
           idx_v, cbuf, band_v, hdrin, cin, sc_b, rowbuf,
           hdr_sp, clist_sp, sem_w):
    core = lax.axis_index("c")
    sub = lax.axis_index("s")
    slab_base = sub * SLAB
    pltpu.sync_copy(idx_hbm.at[pl.ds(slab_base, SLAB)], idx_v)

    def compact(dst, x, mask, n):
        mi = jnp.where(mask, 1, 0)
        s = plsc.cumsum(mi)
        plsc.store_scatter(dst, [s - 1 + n], x, mask=mask)
        return n + s[15]

    def scan(match_fn):
        def body(r, n):
            c = idx_v[pl.ds(r * 16, 16)]
            packed = (c << 10) | (lax.iota(jnp.int32, 16) + r * 16)
            return compact(cbuf, packed, match_fn(c), n)
        return lax.fori_loop(0, SLAB // 16, body, 0)

    @pl.when(core == 0)
    def _():
        n = scan(lambda c: c >= TAIL_START)

        def tgroup(t, carry):
            svec = cbuf[pl.ds(t * 16, 16)]
            for j in range(16):
                @pl.when(t * 16 + j < n)
                def _():
                    pltpu.async_copy(
                        tail_hbm.at[(svec[j] >> 10) - TAIL_START],
                        rowbuf.at[j], sem_w,
                    )
            for j in range(16):
                @pl.when(t * 16 + j < n)
                def _():
                    pltpu.make_async_copy(tail_hbm.at[0], rowbuf.at[0], sem_w).wait()
            for j in range(16):
                @pl.when(t * 16 + j < n)
                def _():
                    pltpu.async_copy(
                        rowbuf.at[j],
                        out_hbm.at[slab_base + (svec[j] & (SLAB - 1))], sem_w,
                    )
            for j in range(16):
                @pl.when(t * 16 + j < n)
                def _():
                    pltpu.make_async_copy(rowbuf.at[0], out_hbm.at[0], sem_w).wait()
            return carry

        lax.fori_loop(0, (n + 15) // 16, tgroup, 0)

    n_my = 31 - core

    def one_pass(i, carry):
        g = core + 2 * i
        pltpu.sync_copy(
            tt_hbm.at[:, pl.ds(pl.multiple_of(g * W + sub * BAND, 128), BAND)],
            band_v,
        )
        n = scan(lambda c, g=g: (c >> 14) == g)
        hdrin[pl.ds(0, 16)] = jnp.full((16,), n, jnp.int32)
        pltpu.sync_copy(
            hdrin.at[pl.ds(0, 128)],
            hdr_sp.at[pl.ds(pl.multiple_of(sub * 128, 128), 128)],
        )

        def push(u, carry2):
            pltpu.sync_copy(
                cbuf.at[pl.ds(u * 128, 128)],
                clist_sp.at[pl.ds(pl.multiple_of(sub * SLAB + u * 128, 128), 128)],
            )
            return carry2
        lax.fori_loop(0, (n + 127) // 128, push, 0)
        plsc.subcore_barrier()

        pltpu.sync_copy(hdr_sp, hdrin)
        my_gb = g * NUM_SUBCORES + sub

        def from_src(src, carry3):
            hv = hdrin[pl.ds(src * 128, 16)]
            n_src = hv[0]
            src_base = src * SLAB

            def unit(u, carry4):
                pltpu.sync_copy(
                    clist_sp.at[pl.ds(pl.multiple_of(src * SLAB + u * 128, 128), 128)],
                    cin,
                )
                def filt(q, m):
                    cv = cin[pl.ds(q * 16, 16)]
                    laneg = lax.iota(jnp.int32, 16) + (u * 128 + q * 16)
                    mask = ((cv >> 20) == my_gb) & (laneg < n_src)
                    return compact(sc_b, cv, mask, m)
                m = lax.fori_loop(0, 8, filt, 0)

                def group(t, carry5):
                    svec = sc_b[pl.ds(t * 16, 16)]
                    for j in range(16):
                        cb = (svec[j] >> 10) & (BAND - 1)
                        for q in range(4):
                            rv = plsc.load_gather(
                                band_v,
                                [lax.iota(jnp.int32, 16) + 16 * q,
                                 jnp.full((16,), cb)],
                            )
                            rowbuf[j, pl.ds(16 * q, 16)] = rv
                    for j in range(16):
                        @pl.when(t * 16 + j < m)
                        def _():
                            pltpu.async_copy(
                                rowbuf.at[j],
                                out_hbm.at[src_base + (svec[j] & (SLAB - 1))],
                                sem_w,
                            )
                    for j in range(16):
                        @pl.when(t * 16 + j < m)
                        def _():
                            pltpu.make_async_copy(
                                rowbuf.at[0], out_hbm.at[0], sem_w
                            ).wait()
                    return carry5

                lax.fori_loop(0, (m + 15) // 16, group, 0)
                return carry4

            lax.fori_loop(0, (n_src + 127) // 128, unit, 0)
            return carry3

        lax.fori_loop(0, NUM_SUBCORES, from_src, 0)
        plsc.subcore_barrier()
        return carry

    lax.fori_loop(0, n_my, one_pass, 0)


def kernel(x, table):
    tail = lax.slice_in_dim(table, TAIL_START, N_ROWS, axis=0)
    return _embed(x.astype(jnp.int32), table.T, tail)


# speedup vs baseline: 4.5441x; 2.2977x over previous
import functools

import jax
import jax.numpy as jnp
from jax import lax
from jax.experimental import pallas as pl
from jax.experimental.pallas import tpu as pltpu
from jax.experimental.pallas import tpu_sc as plsc

N_ROWS = 1000001
BATCH = 16384
EMBED_DIM = 64
NUM_SUBCORES = 16
SLAB = BATCH // NUM_SUBCORES
W = 16384
BAND = W // NUM_SUBCORES
N_STRIPES = 61
TAIL_START = N_STRIPES * W

_mesh = plsc.VectorSubcoreMesh(core_axis_name="c", subcore_axis_name="s")


@functools.partial(
    pl.kernel,
    mesh=_mesh,
    out_type=jax.ShapeDtypeStruct((BATCH, EMBED_DIM), jnp.float32),
    scratch_types=[
        pltpu.VMEM((SLAB,), jnp.int32),
        pltpu.VMEM((SLAB,), jnp.int32),
        pltpu.VMEM((EMBED_DIM, BAND), jnp.float32),
        pltpu.VMEM((NUM_SUBCORES * 128,), jnp.int32),
        pltpu.VMEM((128,), jnp.int32),
        pltpu.VMEM((NUM_SUBCORES * 128,), jnp.int32),
        pltpu.VMEM((BATCH + 16,), jnp.int32),
        pltpu.VMEM((16, EMBED_DIM), jnp.float32),
        pltpu.VMEM_SHARED((NUM_SUBCORES * 128,), jnp.int32),
        pltpu.VMEM_SHARED((NUM_SUBCORES * SLAB,), jnp.int32),
        pltpu.SemaphoreType.DMA,
    ],
    compiler_params=pltpu.CompilerParams(use_tc_tiling_on_sc=True, needs_layout_passes=False),
)
def _embed(idx_hbm, tt_hbm, tail_hbm, out_hbm,
           idx_v, cbuf, band_v, hdrin, cin, allin, sc_b, rowbuf,
           hdr_sp, clist_sp, sem_w):
    core = lax.axis_index("c")
    sub = lax.axis_index("s")
    slab_base = sub * SLAB
    pltpu.sync_copy(idx_hbm.at[pl.ds(slab_base, SLAB)], idx_v)

    def compact(dst, x, mask, n):
        mi = jnp.where(mask, 1, 0)
        s = plsc.cumsum(mi)
        plsc.store_scatter(dst, [s - 1 + n], x, mask=mask)
        return n + s[15]

    def scan(match_fn):
        def body(r, n):
            c = idx_v[pl.ds(r * 16, 16)]
            packed = (c << 10) | (lax.iota(jnp.int32, 16) + r * 16)
            return compact(cbuf, packed, match_fn(c), n)
        return lax.fori_loop(0, SLAB // 16, body, 0)

    @pl.when(core == 0)
    def _():
        n = scan(lambda c: c >= TAIL_START)

        def tgroup(t, carry):
            svec = cbuf[pl.ds(t * 16, 16)]
            for j in range(16):
                @pl.when(t * 16 + j < n)
                def _():
                    pltpu.async_copy(
                        tail_hbm.at[(svec[j] >> 10) - TAIL_START],
                        rowbuf.at[j], sem_w,
                    )
            for j in range(16):
                @pl.when(t * 16 + j < n)
                def _():
                    pltpu.make_async_copy(tail_hbm.at[0], rowbuf.at[0], sem_w).wait()
            for j in range(16):
                @pl.when(t * 16 + j < n)
                def _():
                    pltpu.async_copy(
                        rowbuf.at[j],
                        out_hbm.at[slab_base + (svec[j] & (SLAB - 1))], sem_w,
                    )
            for j in range(16):
                @pl.when(t * 16 + j < n)
                def _():
                    pltpu.make_async_copy(rowbuf.at[0], out_hbm.at[0], sem_w).wait()
            return carry

        lax.fori_loop(0, (n + 15) // 16, tgroup, 0)

    n_my = 31 - core

    def one_pass(i, carry):
        g = core + 2 * i
        pltpu.sync_copy(
            tt_hbm.at[:, pl.ds(pl.multiple_of(g * W + sub * BAND, 128), BAND)],
            band_v,
        )
        n = scan(lambda c, g=g: (c >> 14) == g)
        hdrin[pl.ds(0, 16)] = jnp.full((16,), n, jnp.int32)
        pltpu.sync_copy(
            hdrin.at[pl.ds(0, 128)],
            hdr_sp.at[pl.ds(pl.multiple_of(sub * 128, 128), 128)],
        )

        def push(u, carry2):
            pltpu.sync_copy(
                cbuf.at[pl.ds(u * 128, 128)],
                clist_sp.at[pl.ds(pl.multiple_of((u * 16 + sub) * 128, 128), 128)],
            )
            return carry2
        lax.fori_loop(0, (n + 127) // 128, push, 0)
        plsc.subcore_barrier()

        pltpu.sync_copy(hdr_sp, hdrin)
        pltpu.sync_copy(clist_sp.at[pl.ds(0, NUM_SUBCORES * 128)], allin)
        my_gb = g * NUM_SUBCORES + sub

        def repack(cv, src):
            cb = (cv >> 10) & (BAND - 1)
            gpos = src * SLAB + (cv & (SLAB - 1))
            return (cb << 14) | gpos

        def from_src(src, m0):
            hv = hdrin[pl.ds(src * 128, 16)]
            n_src = hv[0]

            def filt0(q, m):
                cv = allin[pl.ds(src * 128 + q * 16, 16)]
                laneg = lax.iota(jnp.int32, 16) + q * 16
                mask = ((cv >> 20) == my_gb) & (laneg < n_src)
                return compact(sc_b, repack(cv, src), mask, m)
            m1 = lax.fori_loop(0, 8, filt0, m0)

            def unit(u2, m):
                u = u2 + 1
                pltpu.sync_copy(
                    clist_sp.at[pl.ds(pl.multiple_of((u * 16 + src) * 128, 128), 128)],
                    cin,
                )
                def filt(q, m2):
                    cv = cin[pl.ds(q * 16, 16)]
                    laneg = lax.iota(jnp.int32, 16) + (u * 128 + q * 16)
                    mask = ((cv >> 20) == my_gb) & (laneg < n_src)
                    return compact(sc_b, repack(cv, src), mask, m2)
                return lax.fori_loop(0, 8, filt, m)

            extra = jnp.maximum((n_src - 1) // 128, 0)
            return lax.fori_loop(0, extra, unit, m1)

        m = lax.fori_loop(0, NUM_SUBCORES, from_src, 0)

        def group(t, carry5):
            svec = sc_b[pl.ds(t * 16, 16)]
            for j in range(16):
                cb = svec[j] >> 14
                for q in range(4):
                    rv = plsc.load_gather(
                        band_v,
                        [lax.iota(jnp.int32, 16) + 16 * q,
                         jnp.full((16,), cb)],
                    )
                    rowbuf[j, pl.ds(16 * q, 16)] = rv
            for j in range(16):
                @pl.when(t * 16 + j < m)
                def _():
                    pltpu.async_copy(
                        rowbuf.at[j],
                        out_hbm.at[svec[j] & (BATCH - 1)],
                        sem_w,
                    )
            for j in range(16):
                @pl.when(t * 16 + j < m)
                def _():
                    pltpu.make_async_copy(rowbuf.at[0], out_hbm.at[0], sem_w).wait()
            return carry5

        lax.fori_loop(0, (m + 15) // 16, group, 0)
        plsc.subcore_barrier()
        return carry

    lax.fori_loop(0, n_my, one_pass, 0)


def kernel(x, table):
    tail = lax.slice_in_dim(table, TAIL_START, N_ROWS, axis=0)
    return _embed(x.astype(jnp.int32), table.T, tail)
